# Initial kernel scaffold; baseline (speedup 1.0000x reference)
import jax
import jax.numpy as jnp
from jax.experimental import pallas as pl


def kernel(preprocessed_inputs, box_encodings, class_predictions_with_background, rpn_box_predictor_features, rpn_features_to_crop):
    raise NotImplementedError("write your pallas kernel here")


# speedup vs baseline: 21.4413x; 21.4413x over previous
import numpy as np
import jax
import jax.numpy as jnp
from jax import lax
from jax.experimental import pallas as pl

_SCALES = (0.25, 0.5, 1.0, 2.0)
_ASPECT_RATIOS = (0.5, 1.0, 2.0)
_ANCHOR_STRIDE = (16, 16)
_MAX_PROPOSALS = 100
_NMS_IOU_THRESHOLD = 0.699999988079
_BASE_ANCHOR_SIZE = 256.0

_ROWS, _COLS = 96, 128


def _anchor_planes(Hf, Wf):
    ys = (np.arange(Hf, dtype=np.float32) + 0.5) * _ANCHOR_STRIDE[0]
    xs = (np.arange(Wf, dtype=np.float32) + 0.5) * _ANCHOR_STRIDE[1]
    sc, ar = np.meshgrid(np.array(_SCALES, np.float32),
                         np.array(_ASPECT_RATIOS, np.float32), indexing='ij')
    sc = sc.reshape(-1)
    ar = ar.reshape(-1)
    ha = sc * _BASE_ANCHOR_SIZE / np.sqrt(ar)
    wa = sc * _BASE_ANCHOR_SIZE * np.sqrt(ar)
    A = ha.shape[0]
    yy, xx = np.meshgrid(ys, xs, indexing='ij')
    ycent = np.repeat(yy.reshape(-1), A)
    xcent = np.repeat(xx.reshape(-1), A)
    hh = np.tile(ha, Hf * Wf)
    ww = np.tile(wa, Hf * Wf)
    shape = (_ROWS, _COLS)
    return (jnp.asarray(ycent.reshape(shape)), jnp.asarray(xcent.reshape(shape)),
            jnp.asarray(hh.reshape(shape)), jnp.asarray(ww.reshape(shape)))


def _nms_body(tyr, txr, thr_, twr, cbr, cfr, yar, xar, har, war, out_ref):
    H = 512.0
    W = 512.0
    ya = yar[:]
    xa = xar[:]
    ha = har[:]
    wa = war[:]
    ty = tyr[:] / 10.0
    tx = txr[:] / 10.0
    th = thr_[:] / 5.0
    tw = twr[:] / 5.0
    ycenter = ty * ha + ya
    xcenter = tx * wa + xa
    h = jnp.exp(th) * ha
    w = jnp.exp(tw) * wa
    ymin = jnp.clip(ycenter - h / 2.0, 0.0, H)
    xmin = jnp.clip(xcenter - w / 2.0, 0.0, W)
    ymax = jnp.clip(ycenter + h / 2.0, 0.0, H)
    xmax = jnp.clip(xcenter + w / 2.0, 0.0, W)

    cb = cbr[:]
    cf = cfr[:]
    mx = jnp.maximum(cb, cf)
    eb = jnp.exp(cb - mx)
    ef = jnp.exp(cf - mx)
    scores0 = ef / (eb + ef)

    area = jnp.maximum(ymax - ymin, 0.0) * jnp.maximum(xmax - xmin, 0.0)
    lin = (lax.broadcasted_iota(jnp.int32, (_ROWS, _COLS), 0) * _COLS
           + lax.broadcasted_iota(jnp.int32, (_ROWS, _COLS), 1))
    lane = lax.broadcasted_iota(jnp.int32, (1, _COLS), 1)
    thr = jnp.float32(_NMS_IOU_THRESHOLD)
    inv = jnp.float32(1.0 / 512.0)

    def step(t, scores):
        m = jnp.max(scores)
        cand = jnp.where(scores == m, lin, jnp.int32(2 ** 30))
        imin = jnp.min(cand)
        sel = lin == imin
        by0 = jnp.max(jnp.where(sel, ymin, -1.0))
        by1 = jnp.max(jnp.where(sel, xmin, -1.0))
        by2 = jnp.max(jnp.where(sel, ymax, -1.0))
        by3 = jnp.max(jnp.where(sel, xmax, -1.0))
        iy1 = jnp.maximum(by0, ymin)
        ix1 = jnp.maximum(by1, xmin)
        iy2 = jnp.minimum(by2, ymax)
        ix2 = jnp.minimum(by3, xmax)
        inter = jnp.maximum(iy2 - iy1, 0.0) * jnp.maximum(ix2 - ix1, 0.0)
        area_a = jnp.maximum(by2 - by0, 0.0) * jnp.maximum(by3 - by1, 0.0)
        union = area_a + area - inter
        iou = inter / jnp.maximum(union, 1e-8)
        suppress = (iou > thr) | sel
        new_scores = jnp.where(suppress, jnp.float32(-1e9), scores)
        valid = m > 0.0
        row = (jnp.where(lane == 0, by0, 0.0) + jnp.where(lane == 1, by1, 0.0)
               + jnp.where(lane == 2, by2, 0.0) + jnp.where(lane == 3, by3, 0.0))
        row = jnp.where(valid, row, 0.0) * inv
        out_ref[pl.ds(t, 1), :] = row
        return new_scores

    lax.fori_loop(0, _MAX_PROPOSALS, step, scores0)


def kernel(preprocessed_inputs, box_encodings, class_predictions_with_background,
           rpn_box_predictor_features, rpn_features_to_crop):
    del preprocessed_inputs, rpn_box_predictor_features, rpn_features_to_crop
    Hf = Wf = 32
    shape = (_ROWS, _COLS)
    enc = box_encodings[0]
    tyc = enc[:, 0].reshape(shape)
    txc = enc[:, 1].reshape(shape)
    thc = enc[:, 2].reshape(shape)
    twc = enc[:, 3].reshape(shape)
    cls = class_predictions_with_background[0]
    cb = cls[:, 0].reshape(shape)
    cf = cls[:, 1].reshape(shape)
    ya, xa, ha, wa = _anchor_planes(Hf, Wf)
    out = pl.pallas_call(
        _nms_body,
        out_shape=jax.ShapeDtypeStruct((_MAX_PROPOSALS, _COLS), jnp.float32),
    )(tyc, txc, thc, twc, cb, cf, ya, xa, ha, wa)
    return out[:, :4][None]
